# baseline (speedup 1.0000x reference)
import functools

import jax
import jax.numpy as jnp
from jax import lax
from jax.experimental import pallas as pl
from jax.experimental.pallas import tpu as pltpu

_C = 4
_THRESH = 0.5
_VARIANCE = 0.1
_NEGPOS = 3
_ALPHA = 0.1
_PROB_TH = 0.3


def _mbl_kernel(targets_ref, loc_ref, cnf_ref, reg_ref, priors_ref,
                sums_ref, cm_ref, *, n_priors, rows, num_objs):
    f32 = jnp.float32
    i32 = jnp.int32
    R, NP = rows, n_priors

    tr = targets_ref[0]

    def tfield(t, c):
        k = t * 6 + c
        return tr[:, k:k + 1]

    pcx = priors_ref[0:1]
    pcy = priors_ref[1:2]
    pw = priors_ref[2:3]
    ph = priors_ref[3:4]
    px1 = pcx - pw / 2.0
    py1 = pcy - ph / 2.0
    px2 = pcx + pw / 2.0
    py2 = pcy + ph / 2.0
    area_p = (px2 - px1) * (py2 - py1)

    pidx = lax.broadcasted_iota(i32, (R, NP), 1)

    ov = []
    for t in range(num_objs):
        tx1, ty1 = tfield(t, 0), tfield(t, 1)
        tx2, ty2 = tfield(t, 2), tfield(t, 3)
        ix1 = jnp.maximum(tx1, px1)
        iy1 = jnp.maximum(ty1, py1)
        ix2 = jnp.minimum(tx2, px2)
        iy2 = jnp.minimum(ty2, py2)
        iw = jnp.clip(ix2 - ix1, 0.0)
        ih = jnp.clip(iy2 - iy1, 0.0)
        inter = iw * ih
        area_t = (tx2 - tx1) * (ty2 - ty1)
        ov.append(inter / (area_t + area_p - inter))

    bto = ov[0]
    bti = jnp.zeros((R, NP), i32)
    for t in range(1, num_objs):
        upd = ov[t] > bto
        bto = jnp.where(upd, ov[t], bto)
        bti = jnp.where(upd, t, bti)

    for t in range(num_objs):
        mt = jnp.max(ov[t], axis=1, keepdims=True)
        jid = jnp.min(jnp.where(ov[t] == mt, pidx, NP), axis=1, keepdims=True)
        force = pidx == jid
        bto = jnp.where(force, 2.0, bto)
        bti = jnp.where(force, t, bti)

    ctx = jnp.zeros((R, NP), f32)
    cty = jnp.zeros((R, NP), f32)
    conf = jnp.zeros((R, NP), f32)
    regt = jnp.zeros((R, NP), f32)
    for t in range(num_objs):
        st = bti == t
        ctx = jnp.where(st, (tfield(t, 0) + tfield(t, 2)) / 2.0, ctx)
        cty = jnp.where(st, (tfield(t, 1) + tfield(t, 3)) / 2.0, cty)
        conf = jnp.where(st, tfield(t, 4), conf)
        regt = jnp.where(st, tfield(t, 5), regt)
    conf = jnp.where(bto < _THRESH, 0.0, conf)
    pos = conf > 0.0
    posf = pos.astype(f32)
    npos_f = jnp.sum(posf, axis=1, keepdims=True)

    ltx = (ctx - pcx) / (_VARIANCE * pw)
    lty = (cty - pcy) / (_VARIANCE * ph)
    dx = loc_ref[0, 0] - ltx
    dy = loc_ref[0, 1] - lty
    sl1 = (jnp.where(jnp.abs(dx) < 1.0, 0.5 * dx * dx, jnp.abs(dx) - 0.5)
           + jnp.where(jnp.abs(dy) < 1.0, 0.5 * dy * dy, jnp.abs(dy) - 0.5))

    x = [cnf_ref[0, c] for c in range(_C)]
    m = jnp.maximum(jnp.maximum(x[0], x[1]), jnp.maximum(x[2], x[3]))
    e = [jnp.exp(xc - m) for xc in x]
    s = e[0] + e[1] + e[2] + e[3]
    lse = jnp.log(s) + m
    gathered = jnp.zeros((R, NP), f32)
    for c in range(_C):
        gathered = jnp.where(conf == float(c), x[c], gathered)
    v = jnp.where(pos, 0.0, lse - gathered)

    k = jnp.minimum(_NEGPOS * npos_f, float(NP - 1))
    bits = lax.bitcast_convert_type(v, i32)
    maxbits = jnp.max(bits, axis=1, keepdims=True)

    def body(_, lohi):
        lo, hi = lohi
        mid = lo + (hi - lo + 1) // 2
        cnt = jnp.sum((bits >= mid).astype(f32), axis=1, keepdims=True)
        ok = cnt >= k
        return (jnp.where(ok, mid, lo), jnp.where(ok, hi, mid - 1))

    lo, _ = lax.fori_loop(0, 31, body,
                          (jnp.zeros((R, 1), i32), maxbits))
    sel = jnp.logical_or(pos, bits >= lo)
    selm = sel.astype(f32)

    ll = jnp.sum(sl1 * posf, axis=1, keepdims=True)

    lc = jnp.zeros((R, 1), f32)
    for c in range(_C):
        sm = jnp.where(conf == float(c), 1.0 - _ALPHA + _ALPHA / _C,
                       _ALPHA / _C)
        bce = (jnp.maximum(x[c], 0.0) - x[c] * sm
               + jnp.log1p(jnp.exp(-jnp.abs(x[c]))))
        lc = lc + jnp.sum(bce * selm, axis=1, keepdims=True)

    dr = reg_ref[0] - regt
    slr = jnp.where(jnp.abs(dr) < 1.0, 0.5 * dr * dr, jnp.abs(dr) - 0.5)
    lr = jnp.sum(slr * posf, axis=1, keepdims=True)

    best = x[0]
    pred = jnp.zeros((R, NP), f32)
    for c in range(1, _C):
        upd = x[c] > best
        best = jnp.where(upd, x[c], best)
        pred = jnp.where(upd, float(c), pred)

    tm = [selm * (conf == float(c)).astype(f32) for c in range(_C)]
    pm = [(pred == float(d)).astype(f32) for d in range(_C)]
    i0 = lax.broadcasted_iota(i32, (_C, _C), 0)
    i1 = lax.broadcasted_iota(i32, (_C, _C), 1)
    cm_step = jnp.zeros((_C, _C), f32)
    for c in range(_C):
        for d in range(_C):
            s_cd = jnp.sum(tm[c] * pm[d])
            cm_step = cm_step + jnp.where(
                jnp.logical_and(i0 == c, i1 == d), s_cd, 0.0)

    hasb = jnp.max((conf == 1.0).astype(f32), axis=1, keepdims=True)
    p1 = e[1] / s
    predb = jnp.max((p1 > _PROB_TH).astype(f32), axis=1, keepdims=True)
    tp = hasb * predb

    lane8 = lax.broadcasted_iota(i32, (1, 8), 1)
    vec = (jnp.where(lane8 == 0, ll, 0.0)
           + jnp.where(lane8 == 1, lc, 0.0)
           + jnp.where(lane8 == 2, lr, 0.0)
           + jnp.where(lane8 == 3, npos_f, 0.0)
           + jnp.where(lane8 == 4, tp, 0.0)
           + jnp.where(lane8 == 5, predb, 0.0)
           + jnp.where(lane8 == 6, hasb, 0.0))
    sums_ref[0] = vec
    cm_ref[0] = cm_step


def kernel(loc_data, cnf_data, reg_data, targets, priors):
    bs, n_priors, _ = cnf_data.shape
    num_objs = targets.shape[1]
    R = 16
    ns = bs // R
    locT = (loc_data.transpose(0, 2, 1)
            .reshape(ns, R, 2, n_priors).transpose(0, 2, 1, 3))
    cnfT = (cnf_data.transpose(0, 2, 1)
            .reshape(ns, R, _C, n_priors).transpose(0, 2, 1, 3))
    regT = reg_data.reshape(ns, R, n_priors)
    priT = priors.T
    tgtT = targets.reshape(ns, R, num_objs * 6)

    sums, cm = pl.pallas_call(
        functools.partial(_mbl_kernel, n_priors=n_priors, rows=R,
                          num_objs=num_objs),
        grid=(ns,),
        in_specs=[
            pl.BlockSpec((1, R, num_objs * 6), lambda s: (s, 0, 0)),
            pl.BlockSpec((1, 2, R, n_priors), lambda s: (s, 0, 0, 0)),
            pl.BlockSpec((1, _C, R, n_priors), lambda s: (s, 0, 0, 0)),
            pl.BlockSpec((1, R, n_priors), lambda s: (s, 0, 0)),
            pl.BlockSpec((4, n_priors), lambda s: (0, 0)),
        ],
        out_specs=[
            pl.BlockSpec((1, R, 8), lambda s: (s, 0, 0)),
            pl.BlockSpec((1, _C, _C), lambda s: (s, 0, 0)),
        ],
        out_shape=[
            jax.ShapeDtypeStruct((ns, R, 8), jnp.float32),
            jax.ShapeDtypeStruct((ns, _C, _C), jnp.float32),
        ],
        compiler_params=pltpu.CompilerParams(
            dimension_semantics=("parallel",)),
    )(tgtT, locT, cnfT, regT, priT)

    s = sums.reshape(bs, 8).sum(axis=0)
    loss_l, loss_c, loss_r = s[0], s[1], s[2]
    Nf, TP, denp, denh = s[3], s[4], s[5], s[6]
    cm = cm.sum(axis=0)
    col = cm.sum(axis=0)
    row = cm.sum(axis=1)
    diag = jnp.diagonal(cm)
    acc = jnp.where(col > 0, diag / jnp.maximum(col, 1e-9), 0.0)
    rec = jnp.where(row > 0, diag / jnp.maximum(row, 1e-9), 0.0)
    eventPre = jnp.where(denp > 0, TP / jnp.maximum(denp, 1.0), 0.0)
    eventRec = jnp.where(denh > 0, TP / jnp.maximum(denh, 1.0), 0.0)
    return (loss_l / Nf, loss_c / Nf, loss_r / Nf,
            jnp.stack([acc, rec]), jnp.stack([eventPre, eventRec]))


# speedup vs baseline: 1.0731x; 1.0731x over previous
import functools

import jax
import jax.numpy as jnp
from jax import lax
from jax.experimental import pallas as pl
from jax.experimental.pallas import tpu as pltpu

_C = 4
_THRESH = 0.5
_VARIANCE = 0.1
_NEGPOS = 3
_ALPHA = 0.1
_PROB_TH = 0.3


def _mbl_kernel(targets_ref, loc_ref, cnf_ref, reg_ref, priors_ref,
                sums_ref, cm_ref, *, n_priors, rows, num_objs):
    f32 = jnp.float32
    i32 = jnp.int32
    R, NP = rows, n_priors

    tr = targets_ref[0]

    def tfield(t, c):
        k = t * 6 + c
        return tr[:, k:k + 1]

    pcx = priors_ref[0:1]
    pcy = priors_ref[1:2]
    pw = priors_ref[2:3]
    ph = priors_ref[3:4]
    px1 = pcx - pw / 2.0
    py1 = pcy - ph / 2.0
    px2 = pcx + pw / 2.0
    py2 = pcy + ph / 2.0
    area_p = (px2 - px1) * (py2 - py1)

    pidx = lax.broadcasted_iota(i32, (R, NP), 1)

    ov = []
    for t in range(num_objs):
        tx1, ty1 = tfield(t, 0), tfield(t, 1)
        tx2, ty2 = tfield(t, 2), tfield(t, 3)
        ix1 = jnp.maximum(tx1, px1)
        iy1 = jnp.maximum(ty1, py1)
        ix2 = jnp.minimum(tx2, px2)
        iy2 = jnp.minimum(ty2, py2)
        iw = jnp.clip(ix2 - ix1, 0.0)
        ih = jnp.clip(iy2 - iy1, 0.0)
        inter = iw * ih
        area_t = (tx2 - tx1) * (ty2 - ty1)
        ov.append(inter / (area_t + area_p - inter))

    bto = ov[0]
    bti = jnp.zeros((R, NP), i32)
    for t in range(1, num_objs):
        upd = ov[t] > bto
        bto = jnp.where(upd, ov[t], bto)
        bti = jnp.where(upd, t, bti)

    for t in range(num_objs):
        mt = jnp.max(ov[t], axis=1, keepdims=True)
        jid = jnp.min(jnp.where(ov[t] == mt, pidx, NP), axis=1, keepdims=True)
        force = pidx == jid
        bto = jnp.where(force, 2.0, bto)
        bti = jnp.where(force, t, bti)

    mx1 = jnp.zeros((R, NP), f32)
    my1 = jnp.zeros((R, NP), f32)
    mx2 = jnp.zeros((R, NP), f32)
    my2 = jnp.zeros((R, NP), f32)
    conf = jnp.zeros((R, NP), f32)
    regt = jnp.zeros((R, NP), f32)
    for t in range(num_objs):
        st = bti == t
        mx1 = jnp.where(st, tfield(t, 0), mx1)
        my1 = jnp.where(st, tfield(t, 1), my1)
        mx2 = jnp.where(st, tfield(t, 2), mx2)
        my2 = jnp.where(st, tfield(t, 3), my2)
        conf = jnp.where(st, tfield(t, 4), conf)
        regt = jnp.where(st, tfield(t, 5), regt)
    conf = jnp.where(bto < _THRESH, 0.0, conf)
    pos = conf > 0.0
    posf = pos.astype(f32)
    npos_f = jnp.sum(posf, axis=1, keepdims=True)

    ltx = ((mx1 + mx2) / 2.0 - pcx) / (_VARIANCE * pw)
    lty = ((my1 + my2) / 2.0 - pcy) / (_VARIANCE * ph)
    dx = loc_ref[0, 0] - ltx
    dy = loc_ref[0, 1] - lty
    sl1 = (jnp.where(jnp.abs(dx) < 1.0, 0.5 * dx * dx, jnp.abs(dx) - 0.5)
           + jnp.where(jnp.abs(dy) < 1.0, 0.5 * dy * dy, jnp.abs(dy) - 0.5))
    ll = jnp.sum(sl1 * posf, axis=1, keepdims=True)

    x = [cnf_ref[0, c] for c in range(_C)]
    m = jnp.maximum(jnp.maximum(x[0], x[1]), jnp.maximum(x[2], x[3]))
    e = [jnp.exp(xc - m) for xc in x]
    s = e[0] + e[1] + e[2] + e[3]
    lse = jnp.log(s) + m
    gathered = jnp.zeros((R, NP), f32)
    for c in range(_C):
        gathered = jnp.where(conf == float(c), x[c], gathered)
    v = jnp.where(pos, 0.0, lse - gathered)

    npos_i = jnp.sum(pos.astype(i32), axis=1, keepdims=True)
    k = jnp.minimum(_NEGPOS * npos_i, NP - 1)
    bits = lax.bitcast_convert_type(v, i32)
    maxbits = jnp.max(bits, axis=1, keepdims=True)

    def body(_, lohi):
        lo, hi = lohi
        cnt = jnp.sum((bits >= (lo + (hi - lo + 1) // 2)).astype(i32), axis=1, keepdims=True)
        mid = lo + (hi - lo + 1) // 2
        ok = cnt >= k
        return (jnp.where(ok, mid, lo), jnp.where(ok, hi, mid - 1))

    lo, _ = lax.fori_loop(0, 31, body,
                          (jnp.zeros((R, 1), i32), maxbits))
    sel = jnp.logical_or(pos, bits >= lo)
    selm = sel.astype(f32)

    lc = jnp.zeros((R, 1), f32)
    for c in range(_C):
        sm = jnp.where(conf == float(c), 1.0 - _ALPHA + _ALPHA / _C,
                       _ALPHA / _C)
        bce = (jnp.maximum(x[c], 0.0) - x[c] * sm
               + jnp.log1p(jnp.exp(-jnp.abs(x[c]))))
        lc = lc + jnp.sum(bce * selm, axis=1, keepdims=True)

    dr = reg_ref[0] - regt
    slr = jnp.where(jnp.abs(dr) < 1.0, 0.5 * dr * dr, jnp.abs(dr) - 0.5)
    lr = jnp.sum(slr * posf, axis=1, keepdims=True)

    best = x[0]
    pred = jnp.zeros((R, NP), f32)
    for c in range(1, _C):
        upd = x[c] > best
        best = jnp.where(upd, x[c], best)
        pred = jnp.where(upd, float(c), pred)

    tm = [selm * (conf == float(c)).astype(f32) for c in range(_C)]
    pm = [(pred == float(d)).astype(f32) for d in range(_C)]
    i0 = lax.broadcasted_iota(i32, (_C, _C), 0)
    i1 = lax.broadcasted_iota(i32, (_C, _C), 1)
    cm_step = jnp.zeros((_C, _C), f32)
    for c in range(_C):
        for d in range(_C):
            s_cd = jnp.sum(tm[c] * pm[d])
            cm_step = cm_step + jnp.where(
                jnp.logical_and(i0 == c, i1 == d), s_cd, 0.0)

    hasb = jnp.max((conf == 1.0).astype(f32), axis=1, keepdims=True)
    p1 = e[1] / s
    predb = jnp.max((p1 > _PROB_TH).astype(f32), axis=1, keepdims=True)
    tp = hasb * predb

    lane8 = lax.broadcasted_iota(i32, (1, 8), 1)
    vec = (jnp.where(lane8 == 0, ll, 0.0)
           + jnp.where(lane8 == 1, lc, 0.0)
           + jnp.where(lane8 == 2, lr, 0.0)
           + jnp.where(lane8 == 3, npos_f, 0.0)
           + jnp.where(lane8 == 4, tp, 0.0)
           + jnp.where(lane8 == 5, predb, 0.0)
           + jnp.where(lane8 == 6, hasb, 0.0))
    sums_ref[0] = vec
    cm_ref[0] = cm_step


def kernel(loc_data, cnf_data, reg_data, targets, priors):
    bs, n_priors, _ = cnf_data.shape
    num_objs = targets.shape[1]
    R = 8
    ns = bs // R
    locT = (loc_data.transpose(0, 2, 1)
            .reshape(ns, R, 2, n_priors).transpose(0, 2, 1, 3))
    cnfT = (cnf_data.transpose(0, 2, 1)
            .reshape(ns, R, _C, n_priors).transpose(0, 2, 1, 3))
    regT = reg_data.reshape(ns, R, n_priors)
    priT = priors.T
    tgtT = targets.reshape(ns, R, num_objs * 6)

    sums, cm = pl.pallas_call(
        functools.partial(_mbl_kernel, n_priors=n_priors, rows=R,
                          num_objs=num_objs),
        grid=(ns,),
        in_specs=[
            pl.BlockSpec((1, R, num_objs * 6), lambda s: (s, 0, 0)),
            pl.BlockSpec((1, 2, R, n_priors), lambda s: (s, 0, 0, 0)),
            pl.BlockSpec((1, _C, R, n_priors), lambda s: (s, 0, 0, 0)),
            pl.BlockSpec((1, R, n_priors), lambda s: (s, 0, 0)),
            pl.BlockSpec((4, n_priors), lambda s: (0, 0)),
        ],
        out_specs=[
            pl.BlockSpec((1, R, 8), lambda s: (s, 0, 0)),
            pl.BlockSpec((1, _C, _C), lambda s: (s, 0, 0)),
        ],
        out_shape=[
            jax.ShapeDtypeStruct((ns, R, 8), jnp.float32),
            jax.ShapeDtypeStruct((ns, _C, _C), jnp.float32),
        ],
        compiler_params=pltpu.CompilerParams(
            dimension_semantics=("parallel",)),
    )(tgtT, locT, cnfT, regT, priT)

    s = sums.reshape(bs, 8).sum(axis=0)
    loss_l, loss_c, loss_r = s[0], s[1], s[2]
    Nf, TP, denp, denh = s[3], s[4], s[5], s[6]
    cm = cm.sum(axis=0)
    col = cm.sum(axis=0)
    row = cm.sum(axis=1)
    diag = jnp.diagonal(cm)
    acc = jnp.where(col > 0, diag / jnp.maximum(col, 1e-9), 0.0)
    rec = jnp.where(row > 0, diag / jnp.maximum(row, 1e-9), 0.0)
    eventPre = jnp.where(denp > 0, TP / jnp.maximum(denp, 1.0), 0.0)
    eventRec = jnp.where(denh > 0, TP / jnp.maximum(denh, 1.0), 0.0)
    return (loss_l / Nf, loss_c / Nf, loss_r / Nf,
            jnp.stack([acc, rec]), jnp.stack([eventPre, eventRec]))
